# speedup vs baseline: 2.9487x; 2.9487x over previous
import jax
import jax.numpy as jnp
from jax.experimental import pallas as pl

N = 10000
D_IN = 256
D_OUT = 256


def _mm_body(x_ref, w_ref, o_ref):
    o_ref[...] = jnp.dot(x_ref[...], w_ref[...],
                         preferred_element_type=jnp.float32)


def _matmul(x, W):
    M, K = x.shape
    _, Nc = W.shape
    BM = 512
    return pl.pallas_call(
        _mm_body,
        grid=(pl.cdiv(M, BM),),
        in_specs=[pl.BlockSpec((BM, K), lambda i: (i, 0)),
                  pl.BlockSpec((K, Nc), lambda i: (0, 0))],
        out_specs=pl.BlockSpec((BM, Nc), lambda i: (i, 0)),
        out_shape=jax.ShapeDtypeStruct((M, Nc), jnp.float32),
    )(x, W)


def kernel(x, edge_index, W, b):
    src = edge_index[0].astype(jnp.int32)
    dst = edge_index[1].astype(jnp.int32)

    h = _matmul(x, W)

    ones = jnp.ones((src.shape[0],), dtype=jnp.float32)
    deg = jnp.ones((N,), dtype=jnp.float32).at[dst].add(ones)
    dis = jax.lax.rsqrt(deg)

    hs = h * dis[:, None]
    tmp = hs.at[dst].add(hs[src])
    out = dis[:, None] * tmp + b
    return jnp.where(out > 0, out, jnp.expm1(out))
